# baseline (speedup 1.0000x reference)
import functools

import jax
import jax.numpy as jnp
from jax.experimental import pallas as pl
from jax.experimental.pallas import tpu as pltpu
from jax.experimental.pallas import tpu_sc as plsc

_VOCAB_PAD = 128
_EMBED = 512
_SEQ = 20
_WINDOW = 64
_NW = 32
_NCHUNK = 4
_FOLD_BM = 128


def _project_body(t_ref, w_ref, b_ref, o_ref):
    o_ref[...] = (
        jnp.dot(t_ref[...], w_ref[...], preferred_element_type=jnp.float32)
        + b_ref[...]
    )


def _project(table_pad, W, b2d):
    return pl.pallas_call(
        _project_body,
        out_shape=jax.ShapeDtypeStruct((_VOCAB_PAD, _EMBED), jnp.float32),
    )(table_pad, W, b2d)


def _gather(P, idx):
    n = idx.shape[0]
    per_w = n // _NW
    nsteps = per_w // _WINDOW
    mesh = plsc.VectorSubcoreMesh(core_axis_name="core", subcore_axis_name="subcore")

    @functools.partial(
        pl.kernel,
        out_type=jax.ShapeDtypeStruct((n, _EMBED), jnp.float32),
        mesh=mesh,
        scratch_types=[
            pltpu.VMEM((per_w,), jnp.int32),
            pltpu.VMEM((_WINDOW, _EMBED), jnp.float32),
            pltpu.VMEM((_WINDOW, _EMBED), jnp.float32),
            pltpu.SemaphoreType.DMA,
            pltpu.SemaphoreType.DMA,
            pltpu.SemaphoreType.DMA,
            pltpu.SemaphoreType.DMA,
        ],
    )
    def k(p_hbm, i_hbm, o_hbm, idx_v, rows0, rows1, gs0, gs1, ss0, ss1):
        wid = jax.lax.axis_index("subcore") * 2 + jax.lax.axis_index("core")
        base = wid * per_w
        bufs = (rows0, rows1)
        gsems = (gs0, gs1)
        ssems = (ss0, ss1)

        pltpu.sync_copy(i_hbm.at[pl.ds(base, per_w)], idx_v)

        def issue_gather(step, b):
            src = p_hbm.at[idx_v.at[pl.ds(step * _WINDOW, _WINDOW)]]
            pltpu.async_copy(src, bufs[b], gsems[b])

        issue_gather(0, 0)
        issue_gather(1, 1)

        @pl.loop(0, nsteps, step=2)
        def _(g0):
            for b in range(2):
                g = g0 + b
                buf, gsem, ssem = bufs[b], gsems[b], ssems[b]
                pltpu.make_async_copy(
                    p_hbm.at[idx_v.at[pl.ds(0, _WINDOW)]], buf, gsem
                ).wait()
                dst = o_hbm.at[pl.ds(base + g * _WINDOW, _WINDOW)]
                pltpu.async_copy(buf, dst, ssem)
                pltpu.make_async_copy(buf, dst, ssem).wait()
                nxt = jnp.where(g + 2 < nsteps, g + 2, g + 2 - nsteps)
                issue_gather(nxt, b)

        for b in range(2):
            pltpu.make_async_copy(
                p_hbm.at[idx_v.at[pl.ds(0, _WINDOW)]], bufs[b], gsems[b]
            ).wait()

    return k(P, idx)


def _fold_body(x_ref, o_ref):
    o_ref[...] = x_ref[...].reshape(o_ref.shape)


def _fold(x, nb):
    return pl.pallas_call(
        _fold_body,
        grid=(nb // _FOLD_BM,),
        in_specs=[pl.BlockSpec((_FOLD_BM * _SEQ, _EMBED), lambda j: (j, 0))],
        out_specs=pl.BlockSpec((_FOLD_BM, _SEQ, _EMBED), lambda j: (j, 0, 0)),
        out_shape=jax.ShapeDtypeStruct((nb, _SEQ, _EMBED), jnp.float32),
    )(x)


def kernel(indices, table, W, b):
    B, S = indices.shape
    table_pad = jnp.pad(table, ((0, _VOCAB_PAD - table.shape[0]), (0, 0)))
    P = _project(table_pad, W, b.reshape(1, _EMBED))
    idx = indices.reshape(B * S).astype(jnp.int32)
    nb_chunk = B // _NCHUNK
    n_chunk = nb_chunk * S
    outs = []
    for i in range(_NCHUNK):
        rows = _gather(P, jax.lax.slice(idx, (i * n_chunk,), ((i + 1) * n_chunk,)))
        outs.append(_fold(rows, nb_chunk))
    return jnp.concatenate(outs, axis=0)


# speedup vs baseline: 1.2962x; 1.2962x over previous
import functools

import jax
import jax.numpy as jnp
from jax.experimental import pallas as pl
from jax.experimental.pallas import tpu as pltpu
from jax.experimental.pallas import tpu_sc as plsc

_VOCAB_PAD = 128
_EMBED = 512
_SEQ = 20
_WINDOW = 64
_NW = 32
_NCHUNK = 1
_FOLD_BM = 128


def _project_body(t_ref, w_ref, b_ref, o_ref):
    o_ref[...] = (
        jnp.dot(t_ref[...], w_ref[...], preferred_element_type=jnp.float32)
        + b_ref[...]
    )


def _project(table_pad, W, b2d):
    return pl.pallas_call(
        _project_body,
        out_shape=jax.ShapeDtypeStruct((_VOCAB_PAD, _EMBED), jnp.float32),
    )(table_pad, W, b2d)


def _gather(P, idx):
    n = idx.shape[0]
    per_w = n // _NW
    nsteps = per_w // _WINDOW
    mesh = plsc.VectorSubcoreMesh(core_axis_name="core", subcore_axis_name="subcore")

    @functools.partial(
        pl.kernel,
        out_type=jax.ShapeDtypeStruct((n, _EMBED), jnp.float32),
        mesh=mesh,
        scratch_types=[
            pltpu.VMEM((per_w,), jnp.int32),
            pltpu.VMEM((_WINDOW, _EMBED), jnp.float32),
            pltpu.VMEM((_WINDOW, _EMBED), jnp.float32),
            pltpu.SemaphoreType.DMA,
            pltpu.SemaphoreType.DMA,
            pltpu.SemaphoreType.DMA,
            pltpu.SemaphoreType.DMA,
        ],
    )
    def k(p_hbm, i_hbm, o_hbm, idx_v, rows0, rows1, gs0, gs1, ss0, ss1):
        wid = jax.lax.axis_index("subcore") * 2 + jax.lax.axis_index("core")
        base = wid * per_w
        bufs = (rows0, rows1)
        gsems = (gs0, gs1)
        ssems = (ss0, ss1)

        pltpu.sync_copy(i_hbm.at[pl.ds(base, per_w)], idx_v)

        def issue_gather(step, b):
            src = p_hbm.at[idx_v.at[pl.ds(step * _WINDOW, _WINDOW)]]
            pltpu.async_copy(src, bufs[b], gsems[b])

        issue_gather(0, 0)
        issue_gather(1, 1)

        @pl.loop(0, nsteps, step=2)
        def _(g0):
            for b in range(2):
                g = g0 + b
                buf, gsem, ssem = bufs[b], gsems[b], ssems[b]
                pltpu.make_async_copy(
                    p_hbm.at[idx_v.at[pl.ds(0, _WINDOW)]], buf, gsem
                ).wait()
                dst = o_hbm.at[pl.ds(base + g * _WINDOW, _WINDOW)]
                pltpu.async_copy(buf, dst, ssem)
                pltpu.make_async_copy(buf, dst, ssem).wait()
                nxt = jnp.where(g + 2 < nsteps, g + 2, g + 2 - nsteps)
                issue_gather(nxt, b)

        for b in range(2):
            pltpu.make_async_copy(
                p_hbm.at[idx_v.at[pl.ds(0, _WINDOW)]], bufs[b], gsems[b]
            ).wait()

    return k(P, idx)


def _fold_body(x_ref, o_ref):
    o_ref[...] = x_ref[...].reshape(o_ref.shape)


def _fold(x, nb):
    return pl.pallas_call(
        _fold_body,
        grid=(nb // _FOLD_BM,),
        in_specs=[pl.BlockSpec((_FOLD_BM * _SEQ, _EMBED), lambda j: (j, 0))],
        out_specs=pl.BlockSpec((_FOLD_BM, _SEQ, _EMBED), lambda j: (j, 0, 0)),
        out_shape=jax.ShapeDtypeStruct((nb, _SEQ, _EMBED), jnp.float32),
    )(x)


def kernel(indices, table, W, b):
    B, S = indices.shape
    table_pad = jnp.pad(table, ((0, _VOCAB_PAD - table.shape[0]), (0, 0)))
    P = _project(table_pad, W, b.reshape(1, _EMBED))
    idx = indices.reshape(B * S).astype(jnp.int32)
    nb_chunk = B // _NCHUNK
    n_chunk = nb_chunk * S
    outs = []
    for i in range(_NCHUNK):
        rows = _gather(P, jax.lax.slice(idx, (i * n_chunk,), ((i + 1) * n_chunk,)))
        outs.append(_fold(rows, nb_chunk))
    return jnp.concatenate(outs, axis=0)


# speedup vs baseline: 1.2995x; 1.0026x over previous
import functools

import jax
import jax.numpy as jnp
from jax.experimental import pallas as pl
from jax.experimental.pallas import tpu as pltpu
from jax.experimental.pallas import tpu_sc as plsc

_VOCAB_PAD = 128
_EMBED = 512
_SEQ = 20
_WINDOW = 64
_NW = 32
_NCHUNK = 1
_FOLD_BM = 256


def _project_body(t_ref, w_ref, b_ref, o_ref):
    o_ref[...] = (
        jnp.dot(t_ref[...], w_ref[...], preferred_element_type=jnp.float32)
        + b_ref[...]
    )


def _project(table_pad, W, b2d):
    return pl.pallas_call(
        _project_body,
        out_shape=jax.ShapeDtypeStruct((_VOCAB_PAD, _EMBED), jnp.float32),
    )(table_pad, W, b2d)


def _gather(P, idx):
    n = idx.shape[0]
    per_w = n // _NW
    nsteps = per_w // _WINDOW
    mesh = plsc.VectorSubcoreMesh(core_axis_name="core", subcore_axis_name="subcore")

    @functools.partial(
        pl.kernel,
        out_type=jax.ShapeDtypeStruct((n, _EMBED), jnp.float32),
        mesh=mesh,
        scratch_types=[
            pltpu.VMEM((per_w,), jnp.int32),
            pltpu.VMEM((_WINDOW, _EMBED), jnp.float32),
            pltpu.VMEM((_WINDOW, _EMBED), jnp.float32),
            pltpu.SemaphoreType.DMA,
            pltpu.SemaphoreType.DMA,
            pltpu.SemaphoreType.DMA,
            pltpu.SemaphoreType.DMA,
        ],
    )
    def k(p_hbm, i_hbm, o_hbm, idx_v, rows0, rows1, gs0, gs1, ss0, ss1):
        wid = jax.lax.axis_index("subcore") * 2 + jax.lax.axis_index("core")
        base = wid * per_w
        bufs = (rows0, rows1)
        gsems = (gs0, gs1)
        ssems = (ss0, ss1)

        pltpu.sync_copy(i_hbm.at[pl.ds(base, per_w)], idx_v)

        def issue_gather(step, b):
            src = p_hbm.at[idx_v.at[pl.ds(step * _WINDOW, _WINDOW)]]
            pltpu.async_copy(src, bufs[b], gsems[b])

        issue_gather(0, 0)
        issue_gather(1, 1)

        @pl.loop(0, nsteps, step=2)
        def _(g0):
            for b in range(2):
                g = g0 + b
                buf, gsem, ssem = bufs[b], gsems[b], ssems[b]
                pltpu.make_async_copy(
                    p_hbm.at[idx_v.at[pl.ds(0, _WINDOW)]], buf, gsem
                ).wait()
                dst = o_hbm.at[pl.ds(base + g * _WINDOW, _WINDOW)]
                pltpu.async_copy(buf, dst, ssem)
                pltpu.make_async_copy(buf, dst, ssem).wait()
                nxt = jnp.where(g + 2 < nsteps, g + 2, g + 2 - nsteps)
                issue_gather(nxt, b)

        for b in range(2):
            pltpu.make_async_copy(
                p_hbm.at[idx_v.at[pl.ds(0, _WINDOW)]], bufs[b], gsems[b]
            ).wait()

    return k(P, idx)


def _fold_body(x_ref, o_ref):
    o_ref[...] = x_ref[...].reshape(o_ref.shape)


def _fold(x, nb):
    return pl.pallas_call(
        _fold_body,
        grid=(nb // _FOLD_BM,),
        in_specs=[pl.BlockSpec((_FOLD_BM * _SEQ, _EMBED), lambda j: (j, 0))],
        out_specs=pl.BlockSpec((_FOLD_BM, _SEQ, _EMBED), lambda j: (j, 0, 0)),
        out_shape=jax.ShapeDtypeStruct((nb, _SEQ, _EMBED), jnp.float32),
    )(x)


def kernel(indices, table, W, b):
    B, S = indices.shape
    table_pad = jnp.pad(table, ((0, _VOCAB_PAD - table.shape[0]), (0, 0)))
    P = _project(table_pad, W, b.reshape(1, _EMBED))
    idx = indices.reshape(B * S).astype(jnp.int32)
    nb_chunk = B // _NCHUNK
    n_chunk = nb_chunk * S
    outs = []
    for i in range(_NCHUNK):
        rows = _gather(P, jax.lax.slice(idx, (i * n_chunk,), ((i + 1) * n_chunk,)))
        outs.append(_fold(rows, nb_chunk))
    return jnp.concatenate(outs, axis=0)


# speedup vs baseline: 1.5297x; 1.1771x over previous
import functools

import jax
import jax.numpy as jnp
from jax.experimental import pallas as pl
from jax.experimental.pallas import tpu as pltpu
from jax.experimental.pallas import tpu_sc as plsc

_VOCAB_PAD = 128
_FEAT_PAD = 256
_EMBED = 512
_SEQ = 20
_WINDOW = 128
_NW = 32
_NCHUNK = 4
_BM = 128


def _gather(table_pad, idx):
    n = idx.shape[0]
    per_w = n // _NW
    nsteps = per_w // _WINDOW
    mesh = plsc.VectorSubcoreMesh(core_axis_name="core", subcore_axis_name="subcore")

    @functools.partial(
        pl.kernel,
        out_type=jax.ShapeDtypeStruct((n, _FEAT_PAD), jnp.float32),
        mesh=mesh,
        scratch_types=[
            pltpu.VMEM((per_w,), jnp.int32),
            pltpu.VMEM((_WINDOW, _FEAT_PAD), jnp.float32),
            pltpu.VMEM((_WINDOW, _FEAT_PAD), jnp.float32),
            pltpu.SemaphoreType.DMA,
            pltpu.SemaphoreType.DMA,
            pltpu.SemaphoreType.DMA,
            pltpu.SemaphoreType.DMA,
        ],
    )
    def k(t_hbm, i_hbm, o_hbm, idx_v, rows0, rows1, gs0, gs1, ss0, ss1):
        wid = jax.lax.axis_index("subcore") * 2 + jax.lax.axis_index("core")
        base = wid * per_w
        bufs = (rows0, rows1)
        gsems = (gs0, gs1)
        ssems = (ss0, ss1)

        pltpu.sync_copy(i_hbm.at[pl.ds(base, per_w)], idx_v)

        def issue_gather(step, b):
            src = t_hbm.at[idx_v.at[pl.ds(step * _WINDOW, _WINDOW)]]
            pltpu.async_copy(src, bufs[b], gsems[b])

        issue_gather(0, 0)
        issue_gather(1, 1)

        @pl.loop(0, nsteps, step=2)
        def _(g0):
            for b in range(2):
                g = g0 + b
                buf, gsem, ssem = bufs[b], gsems[b], ssems[b]
                pltpu.make_async_copy(
                    t_hbm.at[idx_v.at[pl.ds(0, _WINDOW)]], buf, gsem
                ).wait()
                dst = o_hbm.at[pl.ds(base + g * _WINDOW, _WINDOW)]
                pltpu.async_copy(buf, dst, ssem)
                pltpu.make_async_copy(buf, dst, ssem).wait()
                nxt = jnp.where(g + 2 < nsteps, g + 2, g + 2 - nsteps)
                issue_gather(nxt, b)

        for b in range(2):
            pltpu.make_async_copy(
                t_hbm.at[idx_v.at[pl.ds(0, _WINDOW)]], bufs[b], gsems[b]
            ).wait()

    return k(table_pad, idx)


def _matmul_fold_body(*refs):
    xs = refs[:_NCHUNK]
    w_ref, b_ref, o_ref = refs[_NCHUNK], refs[_NCHUNK + 1], refs[_NCHUNK + 2]
    spc = pl.num_programs(0) // _NCHUNK
    c = pl.program_id(0) // spc
    for i in range(_NCHUNK):
        @pl.when(c == i)
        def _():
            y = (
                jnp.dot(xs[i][...], w_ref[...], preferred_element_type=jnp.float32)
                + b_ref[...]
            )
            o_ref[...] = y.reshape(o_ref.shape)


def _matmul_fold(xs, W_pad, b2d, nb):
    steps_per_chunk = nb // _NCHUNK // _BM

    def x_index(i):
        def index_map(j):
            local = jnp.clip(j - i * steps_per_chunk, 0, steps_per_chunk - 1)
            return (local, 0)
        return index_map

    return pl.pallas_call(
        _matmul_fold_body,
        grid=(nb // _BM,),
        in_specs=[
            pl.BlockSpec((_BM * _SEQ, _FEAT_PAD), x_index(i))
            for i in range(_NCHUNK)
        ] + [
            pl.BlockSpec((_FEAT_PAD, _EMBED), lambda j: (0, 0)),
            pl.BlockSpec((1, _EMBED), lambda j: (0, 0)),
        ],
        out_specs=pl.BlockSpec((_BM, _SEQ, _EMBED), lambda j: (j, 0, 0)),
        out_shape=jax.ShapeDtypeStruct((nb, _SEQ, _EMBED), jnp.float32),
    )(*xs, W_pad, b2d)


def kernel(indices, table, W, b):
    B, S = indices.shape
    table_pad = jnp.pad(
        table, ((0, _VOCAB_PAD - table.shape[0]), (0, _FEAT_PAD - table.shape[1]))
    )
    W_pad = jnp.pad(W, ((0, _FEAT_PAD - W.shape[0]), (0, 0)))
    idx = indices.reshape(B * S).astype(jnp.int32)
    n_chunk = B * S // _NCHUNK
    xs = [
        _gather(table_pad, jax.lax.slice(idx, (i * n_chunk,), ((i + 1) * n_chunk,)))
        for i in range(_NCHUNK)
    ]
    return _matmul_fold(xs, W_pad, b.reshape(1, _EMBED), B)


# speedup vs baseline: 1.6454x; 1.0757x over previous
import functools

import jax
import jax.numpy as jnp
from jax.experimental import pallas as pl
from jax.experimental.pallas import tpu as pltpu
from jax.experimental.pallas import tpu_sc as plsc

_VOCAB_PAD = 128
_FEAT_PAD = 256
_EMBED = 512
_SEQ = 20
_WINDOW = 128
_NW = 32
_NCHUNK = 4
_BM = 128


def _gather(table_pad, idx):
    n = idx.shape[0]
    per_w = n // _NW
    nsteps = per_w // _WINDOW
    mesh = plsc.VectorSubcoreMesh(core_axis_name="core", subcore_axis_name="subcore")

    @functools.partial(
        pl.kernel,
        out_type=jax.ShapeDtypeStruct((n, _FEAT_PAD), jnp.float32),
        mesh=mesh,
        scratch_types=[
            pltpu.VMEM((per_w,), jnp.int32),
            pltpu.VMEM((_WINDOW, _FEAT_PAD), jnp.float32),
            pltpu.VMEM((_WINDOW, _FEAT_PAD), jnp.float32),
            pltpu.SemaphoreType.DMA,
            pltpu.SemaphoreType.DMA,
            pltpu.SemaphoreType.DMA,
            pltpu.SemaphoreType.DMA,
        ],
    )
    def k(t_hbm, i_hbm, o_hbm, idx_v, rows0, rows1, gs0, gs1, ss0, ss1):
        wid = jax.lax.axis_index("subcore") * 2 + jax.lax.axis_index("core")
        base = wid * per_w
        bufs = (rows0, rows1)
        gsems = (gs0, gs1)
        ssems = (ss0, ss1)

        pltpu.sync_copy(i_hbm.at[pl.ds(base, per_w)], idx_v)

        def issue_gather(step, b):
            src = t_hbm.at[idx_v.at[pl.ds(step * _WINDOW, _WINDOW)]]
            pltpu.async_copy(src, bufs[b], gsems[b])

        issue_gather(0, 0)
        issue_gather(1, 1)

        @pl.loop(0, nsteps, step=2)
        def _(g0):
            for b in range(2):
                g = g0 + b
                buf, gsem, ssem = bufs[b], gsems[b], ssems[b]
                pltpu.make_async_copy(
                    t_hbm.at[idx_v.at[pl.ds(0, _WINDOW)]], buf, gsem
                ).wait()
                dst = o_hbm.at[pl.ds(base + g * _WINDOW, _WINDOW)]
                pltpu.async_copy(buf, dst, ssem)
                pltpu.make_async_copy(buf, dst, ssem).wait()
                nxt = jnp.where(g + 2 < nsteps, g + 2, g + 2 - nsteps)
                issue_gather(nxt, b)

        for b in range(2):
            pltpu.make_async_copy(
                t_hbm.at[idx_v.at[pl.ds(0, _WINDOW)]], bufs[b], gsems[b]
            ).wait()

    return k(table_pad, idx)


def _matmul_fold_body(x_ref, w_ref, b_ref, o_ref):
    y = (
        jnp.dot(x_ref[...], w_ref[...], preferred_element_type=jnp.float32)
        + b_ref[...]
    )
    o_ref[...] = y.reshape(o_ref.shape)


def _matmul_fold_body_aliased(prev_ref, x_ref, w_ref, b_ref, o_ref):
    del prev_ref
    _matmul_fold_body(x_ref, w_ref, b_ref, o_ref)


def _matmul_fold_chunk(prev, x, W_pad, b2d, nb, chunk):
    spc = nb // _NCHUNK // _BM
    x_spec = pl.BlockSpec((_BM * _SEQ, _FEAT_PAD), lambda j: (j, 0))
    w_spec = pl.BlockSpec((_FEAT_PAD, _EMBED), lambda j: (0, 0))
    b_spec = pl.BlockSpec((1, _EMBED), lambda j: (0, 0))
    out_spec = pl.BlockSpec(
        (_BM, _SEQ, _EMBED), lambda j, c=chunk: (c * spc + j, 0, 0)
    )
    out_shape = jax.ShapeDtypeStruct((nb, _SEQ, _EMBED), jnp.float32)
    if prev is None:
        return pl.pallas_call(
            _matmul_fold_body,
            grid=(spc,),
            in_specs=[x_spec, w_spec, b_spec],
            out_specs=out_spec,
            out_shape=out_shape,
        )(x, W_pad, b2d)
    return pl.pallas_call(
        _matmul_fold_body_aliased,
        grid=(spc,),
        in_specs=[pl.BlockSpec(memory_space=pl.ANY), x_spec, w_spec, b_spec],
        out_specs=out_spec,
        out_shape=out_shape,
        input_output_aliases={0: 0},
    )(prev, x, W_pad, b2d)


def kernel(indices, table, W, b):
    B, S = indices.shape
    table_pad = jnp.pad(
        table, ((0, _VOCAB_PAD - table.shape[0]), (0, _FEAT_PAD - table.shape[1]))
    )
    W_pad = jnp.pad(W, ((0, _FEAT_PAD - W.shape[0]), (0, 0)))
    b2d = b.reshape(1, _EMBED)
    idx = indices.reshape(B * S).astype(jnp.int32)
    n_chunk = B * S // _NCHUNK
    xs = [
        _gather(table_pad, jax.lax.slice(idx, (i * n_chunk,), ((i + 1) * n_chunk,)))
        for i in range(_NCHUNK)
    ]
    out = None
    for i in range(_NCHUNK):
        out = _matmul_fold_chunk(out, xs[i], W_pad, b2d, B, i)
    return out


# speedup vs baseline: 2.0597x; 1.2518x over previous
import functools

import jax
import jax.numpy as jnp
from jax.experimental import pallas as pl
from jax.experimental.pallas import tpu as pltpu
from jax.experimental.pallas import tpu_sc as plsc

_VOCAB_PAD = 128
_FEAT_PAD = 256
_EMBED = 512
_SEQ = 20
_WINDOW = 128
_NW = 32
_NCHUNK = 4
_BM = 128


def _gather(table_pad, idx):
    n = idx.shape[0]
    per_w = n // _NW
    nsteps = per_w // _WINDOW
    mesh = plsc.VectorSubcoreMesh(core_axis_name="core", subcore_axis_name="subcore")

    @functools.partial(
        pl.kernel,
        out_type=jax.ShapeDtypeStruct((n, _FEAT_PAD // 2), jnp.int32),
        mesh=mesh,
        scratch_types=[
            pltpu.VMEM((per_w,), jnp.int32),
            pltpu.VMEM((_WINDOW, _FEAT_PAD // 2), jnp.int32),
            pltpu.VMEM((_WINDOW, _FEAT_PAD // 2), jnp.int32),
            pltpu.SemaphoreType.DMA,
            pltpu.SemaphoreType.DMA,
            pltpu.SemaphoreType.DMA,
            pltpu.SemaphoreType.DMA,
        ],
    )
    def k(t_hbm, i_hbm, o_hbm, idx_v, rows0, rows1, gs0, gs1, ss0, ss1):
        wid = jax.lax.axis_index("subcore") * 2 + jax.lax.axis_index("core")
        base = wid * per_w
        bufs = (rows0, rows1)
        gsems = (gs0, gs1)
        ssems = (ss0, ss1)

        pltpu.sync_copy(i_hbm.at[pl.ds(base, per_w)], idx_v)

        def issue_gather(step, b):
            src = t_hbm.at[idx_v.at[pl.ds(step * _WINDOW, _WINDOW)]]
            pltpu.async_copy(src, bufs[b], gsems[b])

        issue_gather(0, 0)
        issue_gather(1, 1)

        @pl.loop(0, nsteps, step=2)
        def _(g0):
            for b in range(2):
                g = g0 + b
                buf, gsem, ssem = bufs[b], gsems[b], ssems[b]
                pltpu.make_async_copy(
                    t_hbm.at[idx_v.at[pl.ds(0, _WINDOW)]], buf, gsem
                ).wait()
                dst = o_hbm.at[pl.ds(base + g * _WINDOW, _WINDOW)]
                pltpu.async_copy(buf, dst, ssem)
                pltpu.make_async_copy(buf, dst, ssem).wait()
                nxt = jnp.where(g + 2 < nsteps, g + 2, g + 2 - nsteps)
                issue_gather(nxt, b)

        for b in range(2):
            pltpu.make_async_copy(
                t_hbm.at[idx_v.at[pl.ds(0, _WINDOW)]], bufs[b], gsems[b]
            ).wait()

    return k(table_pad, idx)


def _matmul_fold_body(x_ref, we_ref, wo_ref, b_ref, o_ref):
    x32 = x_ref[...]
    x_even = jax.lax.bitcast_convert_type(x32 << 16, jnp.float32)
    x_odd = jax.lax.bitcast_convert_type(
        x32 & jnp.int32(-65536), jnp.float32
    )
    y = (
        jnp.dot(x_even, we_ref[...], preferred_element_type=jnp.float32)
        + jnp.dot(x_odd, wo_ref[...], preferred_element_type=jnp.float32)
        + b_ref[...]
    )
    o_ref[...] = y.reshape(o_ref.shape)


def _matmul_fold_body_aliased(prev_ref, x_ref, we_ref, wo_ref, b_ref, o_ref):
    del prev_ref
    _matmul_fold_body(x_ref, we_ref, wo_ref, b_ref, o_ref)


def _matmul_fold_chunk(prev, x, W_even, W_odd, b2d, nb, chunk):
    spc = nb // _NCHUNK // _BM
    x_spec = pl.BlockSpec((_BM * _SEQ, _FEAT_PAD // 2), lambda j: (j, 0))
    w_spec = pl.BlockSpec((_FEAT_PAD // 2, _EMBED), lambda j: (0, 0))
    b_spec = pl.BlockSpec((1, _EMBED), lambda j: (0, 0))
    out_spec = pl.BlockSpec(
        (_BM, _SEQ, _EMBED), lambda j, c=chunk: (c * spc + j, 0, 0)
    )
    out_shape = jax.ShapeDtypeStruct((nb, _SEQ, _EMBED), jnp.float32)
    if prev is None:
        return pl.pallas_call(
            _matmul_fold_body,
            grid=(spc,),
            in_specs=[x_spec, w_spec, w_spec, b_spec],
            out_specs=out_spec,
            out_shape=out_shape,
        )(x, W_even, W_odd, b2d)
    return pl.pallas_call(
        _matmul_fold_body_aliased,
        grid=(spc,),
        in_specs=[pl.BlockSpec(memory_space=pl.ANY), x_spec, w_spec, w_spec, b_spec],
        out_specs=out_spec,
        out_shape=out_shape,
        input_output_aliases={0: 0},
    )(prev, x, W_even, W_odd, b2d)


def kernel(indices, table, W, b):
    B, S = indices.shape
    table_bf = jnp.pad(
        table, ((0, _VOCAB_PAD - table.shape[0]), (0, _FEAT_PAD - table.shape[1]))
    ).astype(jnp.bfloat16)
    table_pad = jax.lax.bitcast_convert_type(
        table_bf.reshape(_VOCAB_PAD, _FEAT_PAD // 2, 2), jnp.int32
    )
    W_pad = jnp.pad(W, ((0, _FEAT_PAD - W.shape[0]), (0, 0)))
    W_even = W_pad[0::2, :]
    W_odd = W_pad[1::2, :]
    b2d = b.reshape(1, _EMBED)
    idx = indices.reshape(B * S).astype(jnp.int32)
    n_chunk = B * S // _NCHUNK
    xs = [
        _gather(table_pad, jax.lax.slice(idx, (i * n_chunk,), ((i + 1) * n_chunk,)))
        for i in range(_NCHUNK)
    ]
    out = None
    for i in range(_NCHUNK):
        out = _matmul_fold_chunk(out, xs[i], W_even, W_odd, b2d, B, i)
    return out


# speedup vs baseline: 2.6591x; 1.2910x over previous
import functools

import jax
import jax.numpy as jnp
from jax.experimental import pallas as pl
from jax.experimental.pallas import tpu as pltpu
from jax.experimental.pallas import tpu_sc as plsc

_VOCAB_PAD = 128
_FEAT_PAD = 256
_EMBED = 512
_SEQ = 20
_WINDOW = 128
_NW = 32
_NCHUNK = 4
_NSC = 1
_BM = 128


def _gather(table_pad, idx):
    n = idx.shape[0]
    per_w = n // _NW
    nsteps = per_w // _WINDOW
    mesh = plsc.VectorSubcoreMesh(core_axis_name="core", subcore_axis_name="subcore")

    @functools.partial(
        pl.kernel,
        out_type=jax.ShapeDtypeStruct((n, _FEAT_PAD // 2), jnp.int32),
        mesh=mesh,
        scratch_types=[
            pltpu.VMEM((per_w,), jnp.int32),
            pltpu.VMEM((_WINDOW, _FEAT_PAD // 2), jnp.int32),
            pltpu.VMEM((_WINDOW, _FEAT_PAD // 2), jnp.int32),
            pltpu.SemaphoreType.DMA,
            pltpu.SemaphoreType.DMA,
            pltpu.SemaphoreType.DMA,
            pltpu.SemaphoreType.DMA,
        ],
    )
    def k(t_hbm, i_hbm, o_hbm, idx_v, rows0, rows1, gs0, gs1, ss0, ss1):
        wid = jax.lax.axis_index("subcore") * 2 + jax.lax.axis_index("core")
        base = wid * per_w
        bufs = (rows0, rows1)
        gsems = (gs0, gs1)
        ssems = (ss0, ss1)

        pltpu.sync_copy(i_hbm.at[pl.ds(base, per_w)], idx_v)

        def issue_gather(step, b):
            src = t_hbm.at[idx_v.at[pl.ds(step * _WINDOW, _WINDOW)]]
            pltpu.async_copy(src, bufs[b], gsems[b])

        issue_gather(0, 0)
        issue_gather(1, 1)

        @pl.loop(0, nsteps, step=2)
        def _(g0):
            for b in range(2):
                g = g0 + b
                buf, gsem, ssem = bufs[b], gsems[b], ssems[b]
                pltpu.make_async_copy(
                    t_hbm.at[idx_v.at[pl.ds(0, _WINDOW)]], buf, gsem
                ).wait()
                dst = o_hbm.at[pl.ds(base + g * _WINDOW, _WINDOW)]
                pltpu.async_copy(buf, dst, ssem)
                pltpu.make_async_copy(buf, dst, ssem).wait()
                nxt = jnp.where(g + 2 < nsteps, g + 2, g + 2 - nsteps)
                issue_gather(nxt, b)

        for b in range(2):
            pltpu.make_async_copy(
                t_hbm.at[idx_v.at[pl.ds(0, _WINDOW)]], bufs[b], gsems[b]
            ).wait()

    return k(table_pad, idx)


def _proj_table_body(t_ref, w_ref, b_ref, o_ref):
    o_ref[...] = (
        jnp.dot(t_ref[...], w_ref[...], preferred_element_type=jnp.float32)
        + b_ref[...]
    )


def _onehot_body(idx_ref, p_ref, o_ref):
    flat = idx_ref[...]
    oh = (
        jax.lax.broadcasted_iota(jnp.int32, (flat.shape[0], _VOCAB_PAD), 1)
        == flat
    ).astype(jnp.float32)
    y = jnp.dot(oh, p_ref[...], preferred_element_type=jnp.float32)
    o_ref[...] = y.reshape(o_ref.shape)


def _onehot_body_aliased(prev_ref, idx_ref, p_ref, o_ref):
    del prev_ref
    _onehot_body(idx_ref, p_ref, o_ref)


def _onehot_chunk(prev, idx2d, P, nb, chunk):
    spc = nb // _NCHUNK // _BM
    idx_spec = pl.BlockSpec(
        (_BM * _SEQ, 1), lambda j, c=chunk: (c * spc + j, 0)
    )
    p_spec = pl.BlockSpec((_VOCAB_PAD, _EMBED), lambda j: (0, 0))
    out_spec = pl.BlockSpec(
        (_BM, _SEQ, _EMBED), lambda j, c=chunk: (c * spc + j, 0, 0)
    )
    out_shape = jax.ShapeDtypeStruct((nb, _SEQ, _EMBED), jnp.float32)
    if prev is None:
        return pl.pallas_call(
            _onehot_body,
            grid=(spc,),
            in_specs=[idx_spec, p_spec],
            out_specs=out_spec,
            out_shape=out_shape,
        )(idx2d, P)
    return pl.pallas_call(
        _onehot_body_aliased,
        grid=(spc,),
        in_specs=[pl.BlockSpec(memory_space=pl.ANY), idx_spec, p_spec],
        out_specs=out_spec,
        out_shape=out_shape,
        input_output_aliases={0: 0},
    )(prev, idx2d, P)


def _matmul_fold_body(x_ref, we_ref, wo_ref, b_ref, o_ref):
    x32 = x_ref[...]
    x_even = jax.lax.bitcast_convert_type(x32 << 16, jnp.float32)
    x_odd = jax.lax.bitcast_convert_type(
        x32 & jnp.int32(-65536), jnp.float32
    )
    y = (
        jnp.dot(x_even, we_ref[...], preferred_element_type=jnp.float32)
        + jnp.dot(x_odd, wo_ref[...], preferred_element_type=jnp.float32)
        + b_ref[...]
    )
    o_ref[...] = y.reshape(o_ref.shape)


def _matmul_fold_body_aliased(prev_ref, x_ref, we_ref, wo_ref, b_ref, o_ref):
    del prev_ref
    _matmul_fold_body(x_ref, we_ref, wo_ref, b_ref, o_ref)


def _matmul_fold_chunk(prev, x, W_even, W_odd, b2d, nb, chunk):
    spc = nb // _NCHUNK // _BM
    x_spec = pl.BlockSpec((_BM * _SEQ, _FEAT_PAD // 2), lambda j: (j, 0))
    w_spec = pl.BlockSpec((_FEAT_PAD // 2, _EMBED), lambda j: (0, 0))
    b_spec = pl.BlockSpec((1, _EMBED), lambda j: (0, 0))
    out_spec = pl.BlockSpec(
        (_BM, _SEQ, _EMBED), lambda j, c=chunk: (c * spc + j, 0, 0)
    )
    out_shape = jax.ShapeDtypeStruct((nb, _SEQ, _EMBED), jnp.float32)
    if prev is None:
        return pl.pallas_call(
            _matmul_fold_body,
            grid=(spc,),
            in_specs=[x_spec, w_spec, w_spec, b_spec],
            out_specs=out_spec,
            out_shape=out_shape,
        )(x, W_even, W_odd, b2d)
    return pl.pallas_call(
        _matmul_fold_body_aliased,
        grid=(spc,),
        in_specs=[pl.BlockSpec(memory_space=pl.ANY), x_spec, w_spec, w_spec, b_spec],
        out_specs=out_spec,
        out_shape=out_shape,
        input_output_aliases={0: 0},
    )(prev, x, W_even, W_odd, b2d)


def kernel(indices, table, W, b):
    B, S = indices.shape
    table_bf = jnp.pad(
        table, ((0, _VOCAB_PAD - table.shape[0]), (0, _FEAT_PAD - table.shape[1]))
    ).astype(jnp.bfloat16)
    table_pad = jax.lax.bitcast_convert_type(
        table_bf.reshape(_VOCAB_PAD, _FEAT_PAD // 2, 2), jnp.int32
    )
    W_pad = jnp.pad(W, ((0, _FEAT_PAD - W.shape[0]), (0, 0)))
    W_even = W_pad[0::2, :]
    W_odd = W_pad[1::2, :]
    b2d = b.reshape(1, _EMBED)
    idx2d = indices.astype(jnp.int32)
    idx = idx2d.reshape(B * S)
    n_chunk = B * S // _NCHUNK

    table_f32 = jnp.pad(
        table, ((0, _VOCAB_PAD - table.shape[0]), (0, _FEAT_PAD - table.shape[1]))
    )
    P = pl.pallas_call(
        _proj_table_body,
        out_shape=jax.ShapeDtypeStruct((_VOCAB_PAD, _EMBED), jnp.float32),
    )(table_f32, W_pad, b2d)

    sc_chunks = list(range(_NSC))
    tc_chunks = list(range(_NSC, _NCHUNK))
    xs = {
        i: _gather(table_pad, jax.lax.slice(idx, (i * n_chunk,), ((i + 1) * n_chunk,)))
        for i in sc_chunks
    }
    idx_col = idx.reshape(B * S, 1)
    out = None
    for i in tc_chunks:
        out = _onehot_chunk(out, idx_col, P, B, i)
    for i in sc_chunks:
        out = _matmul_fold_chunk(out, xs[i], W_even, W_odd, b2d, B, i)
    return out


# speedup vs baseline: 2.6658x; 1.0025x over previous
import functools

import jax
import jax.numpy as jnp
from jax.experimental import pallas as pl
from jax.experimental.pallas import tpu as pltpu
from jax.experimental.pallas import tpu_sc as plsc

_VOCAB_PAD = 128
_FEAT_PAD = 256
_EMBED = 512
_SEQ = 20
_WINDOW = 128
_NW = 32
_NCHUNK = 4
_NSC = 1
_BM = 128


def _gather(table_pad, idx):
    n = idx.shape[0]
    per_w = n // _NW
    nsteps = per_w // _WINDOW
    mesh = plsc.VectorSubcoreMesh(core_axis_name="core", subcore_axis_name="subcore")

    @functools.partial(
        pl.kernel,
        out_type=jax.ShapeDtypeStruct((n, _FEAT_PAD // 2), jnp.int32),
        mesh=mesh,
        scratch_types=[
            pltpu.VMEM((per_w,), jnp.int32),
            pltpu.VMEM((_WINDOW, _FEAT_PAD // 2), jnp.int32),
            pltpu.VMEM((_WINDOW, _FEAT_PAD // 2), jnp.int32),
            pltpu.SemaphoreType.DMA,
            pltpu.SemaphoreType.DMA,
            pltpu.SemaphoreType.DMA,
            pltpu.SemaphoreType.DMA,
        ],
    )
    def k(t_hbm, i_hbm, o_hbm, idx_v, rows0, rows1, gs0, gs1, ss0, ss1):
        wid = jax.lax.axis_index("subcore") * 2 + jax.lax.axis_index("core")
        base = wid * per_w
        bufs = (rows0, rows1)
        gsems = (gs0, gs1)
        ssems = (ss0, ss1)

        pltpu.sync_copy(i_hbm.at[pl.ds(base, per_w)], idx_v)

        def issue_gather(step, b):
            src = t_hbm.at[idx_v.at[pl.ds(step * _WINDOW, _WINDOW)]]
            pltpu.async_copy(src, bufs[b], gsems[b])

        issue_gather(0, 0)
        issue_gather(1, 1)

        @pl.loop(0, nsteps, step=2)
        def _(g0):
            for b in range(2):
                g = g0 + b
                buf, gsem, ssem = bufs[b], gsems[b], ssems[b]
                pltpu.make_async_copy(
                    t_hbm.at[idx_v.at[pl.ds(0, _WINDOW)]], buf, gsem
                ).wait()
                dst = o_hbm.at[pl.ds(base + g * _WINDOW, _WINDOW)]
                pltpu.async_copy(buf, dst, ssem)
                pltpu.make_async_copy(buf, dst, ssem).wait()
                nxt = jnp.where(g + 2 < nsteps, g + 2, g + 2 - nsteps)
                issue_gather(nxt, b)

        for b in range(2):
            pltpu.make_async_copy(
                t_hbm.at[idx_v.at[pl.ds(0, _WINDOW)]], bufs[b], gsems[b]
            ).wait()

    return k(table_pad, idx)


def _proj_table_body(t_ref, w_ref, b_ref, o_ref):
    o_ref[...] = (
        jnp.dot(t_ref[...], w_ref[...], preferred_element_type=jnp.float32)
        + b_ref[...]
    )


def _onehot_body(idx_ref, ph_ref, pl_ref, o_ref):
    flat = idx_ref[...]
    oh = (
        jax.lax.broadcasted_iota(jnp.int32, (flat.shape[0], _VOCAB_PAD), 1)
        == flat
    ).astype(jnp.bfloat16)
    y = jnp.dot(oh, ph_ref[...], preferred_element_type=jnp.float32) + jnp.dot(
        oh, pl_ref[...], preferred_element_type=jnp.float32
    )
    o_ref[...] = y.reshape(o_ref.shape)


def _onehot_body_aliased(prev_ref, idx_ref, ph_ref, pl_ref, o_ref):
    del prev_ref
    _onehot_body(idx_ref, ph_ref, pl_ref, o_ref)


def _onehot_chunk(prev, idx2d, Ph, Pl, nb, chunk):
    spc = nb // _NCHUNK // _BM
    idx_spec = pl.BlockSpec(
        (_BM * _SEQ, 1), lambda j, c=chunk: (c * spc + j, 0)
    )
    p_spec = pl.BlockSpec((_VOCAB_PAD, _EMBED), lambda j: (0, 0))
    out_spec = pl.BlockSpec(
        (_BM, _SEQ, _EMBED), lambda j, c=chunk: (c * spc + j, 0, 0)
    )
    out_shape = jax.ShapeDtypeStruct((nb, _SEQ, _EMBED), jnp.float32)
    if prev is None:
        return pl.pallas_call(
            _onehot_body,
            grid=(spc,),
            in_specs=[idx_spec, p_spec, p_spec],
            out_specs=out_spec,
            out_shape=out_shape,
        )(idx2d, Ph, Pl)
    return pl.pallas_call(
        _onehot_body_aliased,
        grid=(spc,),
        in_specs=[pl.BlockSpec(memory_space=pl.ANY), idx_spec, p_spec, p_spec],
        out_specs=out_spec,
        out_shape=out_shape,
        input_output_aliases={0: 0},
    )(prev, idx2d, Ph, Pl)


def _matmul_fold_body(x_ref, we_ref, wo_ref, b_ref, o_ref):
    x32 = x_ref[...]
    x_even = jax.lax.bitcast_convert_type(x32 << 16, jnp.float32)
    x_odd = jax.lax.bitcast_convert_type(
        x32 & jnp.int32(-65536), jnp.float32
    )
    y = (
        jnp.dot(x_even, we_ref[...], preferred_element_type=jnp.float32)
        + jnp.dot(x_odd, wo_ref[...], preferred_element_type=jnp.float32)
        + b_ref[...]
    )
    o_ref[...] = y.reshape(o_ref.shape)


def _matmul_fold_body_aliased(prev_ref, x_ref, we_ref, wo_ref, b_ref, o_ref):
    del prev_ref
    _matmul_fold_body(x_ref, we_ref, wo_ref, b_ref, o_ref)


def _matmul_fold_chunk(prev, x, W_even, W_odd, b2d, nb, chunk):
    spc = nb // _NCHUNK // _BM
    x_spec = pl.BlockSpec((_BM * _SEQ, _FEAT_PAD // 2), lambda j: (j, 0))
    w_spec = pl.BlockSpec((_FEAT_PAD // 2, _EMBED), lambda j: (0, 0))
    b_spec = pl.BlockSpec((1, _EMBED), lambda j: (0, 0))
    out_spec = pl.BlockSpec(
        (_BM, _SEQ, _EMBED), lambda j, c=chunk: (c * spc + j, 0, 0)
    )
    out_shape = jax.ShapeDtypeStruct((nb, _SEQ, _EMBED), jnp.float32)
    if prev is None:
        return pl.pallas_call(
            _matmul_fold_body,
            grid=(spc,),
            in_specs=[x_spec, w_spec, w_spec, b_spec],
            out_specs=out_spec,
            out_shape=out_shape,
        )(x, W_even, W_odd, b2d)
    return pl.pallas_call(
        _matmul_fold_body_aliased,
        grid=(spc,),
        in_specs=[pl.BlockSpec(memory_space=pl.ANY), x_spec, w_spec, w_spec, b_spec],
        out_specs=out_spec,
        out_shape=out_shape,
        input_output_aliases={0: 0},
    )(prev, x, W_even, W_odd, b2d)


def kernel(indices, table, W, b):
    B, S = indices.shape
    table_bf = jnp.pad(
        table, ((0, _VOCAB_PAD - table.shape[0]), (0, _FEAT_PAD - table.shape[1]))
    ).astype(jnp.bfloat16)
    table_pad = jax.lax.bitcast_convert_type(
        table_bf.reshape(_VOCAB_PAD, _FEAT_PAD // 2, 2), jnp.int32
    )
    W_pad = jnp.pad(W, ((0, _FEAT_PAD - W.shape[0]), (0, 0)))
    W_even = W_pad[0::2, :]
    W_odd = W_pad[1::2, :]
    b2d = b.reshape(1, _EMBED)
    idx2d = indices.astype(jnp.int32)
    idx = idx2d.reshape(B * S)
    n_chunk = B * S // _NCHUNK

    table_f32 = jnp.pad(
        table, ((0, _VOCAB_PAD - table.shape[0]), (0, _FEAT_PAD - table.shape[1]))
    )
    P = pl.pallas_call(
        _proj_table_body,
        out_shape=jax.ShapeDtypeStruct((_VOCAB_PAD, _EMBED), jnp.float32),
    )(table_f32, W_pad, b2d)

    sc_chunks = list(range(_NSC))
    tc_chunks = list(range(_NSC, _NCHUNK))
    xs = {
        i: _gather(table_pad, jax.lax.slice(idx, (i * n_chunk,), ((i + 1) * n_chunk,)))
        for i in sc_chunks
    }
    Ph = P.astype(jnp.bfloat16)
    Pl = (P - Ph.astype(jnp.float32)).astype(jnp.bfloat16)
    idx_col = idx.reshape(B * S, 1)
    out = None
    for i in tc_chunks:
        out = _onehot_chunk(out, idx_col, Ph, Pl, B, i)
    for i in sc_chunks:
        out = _matmul_fold_chunk(out, xs[i], W_even, W_odd, b2d, B, i)
    return out
